# baseline (speedup 1.0000x reference)
import functools

import jax
import jax.numpy as jnp
from jax import lax
from jax.experimental import pallas as pl
from jax.experimental.pallas import tpu as pltpu
from jax.experimental.pallas import tpu_sc as plsc

_NUM_HASHES = 4
_HASH_VOCAB = 100000
_D = 32
_L = 16
_S = 200

_A_MOD = (7, 11, 13, 17)
_B_MOD = (13, 99996, 99979, 99962)

_R = 2
_C = _R * _S


@functools.cache
def _make_sc_kernel(n_batch):
    info = plsc.get_sparse_core_info()
    nc, ns = info.num_cores, info.num_subcores
    nw = nc * ns
    rows_w = n_batch // nw
    assert rows_w * nw == n_batch and rows_w % (2 * _R) == 0
    n_chunks = rows_w // _R
    per_w = rows_w * _S

    mesh = plsc.VectorSubcoreMesh(core_axis_name="c", subcore_axis_name="s")

    @functools.partial(
        pl.kernel,
        mesh=mesh,
        compiler_params=pltpu.CompilerParams(use_tc_tiling_on_sc=False),
        out_type=jax.ShapeDtypeStruct((n_batch, _S, _D), jnp.float32),
        scratch_types=[
            pltpu.VMEM((2, _C), jnp.int32),
            pltpu.VMEM((2, _NUM_HASHES, _C), jnp.int32),
            pltpu.VMEM((2, _NUM_HASHES, _C, _D), jnp.float32),
            pltpu.VMEM((_C, _D), jnp.float32),
            pltpu.SemaphoreType.DMA,
            pltpu.SemaphoreType.DMA,
            pltpu.SemaphoreType.DMA,
        ],
    )
    def k(tok_hbm, tab_hbm, out_hbm, tok_v, idx_v, rows_v, out_v,
          sem_g0, sem_g1, sem_o):
        i32 = jnp.int32
        wid = lax.axis_index("s") * i32(nc) + lax.axis_index("c")
        wbase = wid * i32(per_w)
        wrow = wid * i32(rows_w)
        sem_g = (sem_g0, sem_g1)

        def stage(g, buf):
            b = i32(buf)
            base = wbase + g * i32(_C)
            pltpu.sync_copy(tok_hbm.at[pl.ds(base, _C)], tok_v.at[b])

            def hash_body(j, carry):
                off = j * i32(_L)
                tv = tok_v[b, pl.ds(off, _L)]
                for i in range(_NUM_HASHES):
                    x = tv * _A_MOD[i] + _B_MOD[i]
                    qf = (x.astype(jnp.float32)
                          * jnp.float32(1.0 / _HASH_VOCAB)
                          + jnp.float32(8388608.0))
                    q = lax.bitcast_convert_type(qf, jnp.int32) - 0x4B000000
                    r = x - q * _HASH_VOCAB
                    r = jnp.where(r < 0, r + _HASH_VOCAB, r)
                    r = jnp.where(r >= _HASH_VOCAB, r - _HASH_VOCAB, r)
                    idx_v[b, i, pl.ds(off, _L)] = r + i * _HASH_VOCAB
                return carry

            lax.fori_loop(i32(0), i32(_C // _L), hash_body, i32(0))
            for i in range(_NUM_HASHES):
                pltpu.make_async_copy(
                    tab_hbm.at[idx_v.at[b, i32(i)]],
                    rows_v.at[b, i32(i)],
                    sem_g[buf],
                ).start()

        def wait_gathers(buf):
            b = i32(buf)
            for i in range(_NUM_HASHES):
                pltpu.make_async_copy(
                    tab_hbm.at[idx_v.at[b, i32(i)]],
                    rows_v.at[b, i32(i)],
                    sem_g[buf],
                ).wait()

        def out_copies(g):
            row = wrow + g * i32(_R)
            return [
                pltpu.make_async_copy(
                    out_v.at[pl.ds(r * _S, _S)],
                    out_hbm.at[row + i32(r)],
                    sem_o,
                )
                for r in range(_R)
            ]

        def accumulate(buf):
            b = i32(buf)

            def acc_body(j, carry):
                t0 = j * i32(_L)
                lo = pl.ds(0, _L)
                hi = pl.ds(_L, _L)
                tv = tok_v[b, pl.ds(t0, _L)]
                svs = [(((tv >> i) & 1) * 2 - 1).astype(jnp.float32)
                       for i in range(_NUM_HASHES)]
                for u in range(_L):
                    t = t0 + i32(u)
                    s0, s1, s2, s3 = (svs[0][u], svs[1][u], svs[2][u],
                                      svs[3][u])
                    out_v[t, lo] = (s0 * rows_v[b, 0, t, lo]
                                    + s1 * rows_v[b, 1, t, lo]
                                    + s2 * rows_v[b, 2, t, lo]
                                    + s3 * rows_v[b, 3, t, lo])
                    out_v[t, hi] = (s0 * rows_v[b, 0, t, hi]
                                    + s1 * rows_v[b, 1, t, hi]
                                    + s2 * rows_v[b, 2, t, hi]
                                    + s3 * rows_v[b, 3, t, hi])
                return carry

            lax.fori_loop(i32(0), i32(_C // _L), acc_body, i32(0))

        def drain_out(g):
            for cp in out_copies(g):
                cp.wait()

        stage(i32(0), 0)

        def pair_body(g2, carry):
            g = g2 * i32(2)
            wait_gathers(0)
            stage(g + i32(1), 1)
            pl.when(g2 > 0)(lambda: drain_out(g - i32(1)))
            accumulate(0)
            for cp in out_copies(g):
                cp.start()
            wait_gathers(1)
            pl.when(g2 < i32(n_chunks // 2 - 1))(
                lambda: stage(g + i32(2), 0))
            drain_out(g)
            accumulate(1)
            for cp in out_copies(g + i32(1)):
                cp.start()
            return carry

        lax.fori_loop(i32(0), i32(n_chunks // 2), pair_body, i32(0))
        drain_out(i32(n_chunks - 1))

    return k


def kernel(token_ids, tables):
    n = token_ids.shape[0] * token_ids.shape[1]
    tok = token_ids.reshape(n).astype(jnp.int32)
    tab = tables.reshape(_NUM_HASHES * _HASH_VOCAB, _D).astype(jnp.float32)
    out = _make_sc_kernel(token_ids.shape[0])(tok, tab)
    return out.astype(jnp.promote_types(jnp.float32, tables.dtype))


# speedup vs baseline: 2.6352x; 2.6352x over previous
import functools

import jax
import jax.numpy as jnp
from jax import lax
from jax.experimental import pallas as pl
from jax.experimental.pallas import tpu as pltpu
from jax.experimental.pallas import tpu_sc as plsc

_NUM_HASHES = 4
_HASH_VOCAB = 100000
_D = 32
_L = 16

_A_MOD = (7, 11, 13, 17)
_B_MOD = (13, 99996, 99979, 99962)

_C = 256


@functools.cache
def _make_sc_kernel(n_tokens):
    info = plsc.get_sparse_core_info()
    nc, ns = info.num_cores, info.num_subcores
    nw = nc * ns
    per_w = n_tokens // nw
    assert per_w * nw == n_tokens and per_w % (2 * _C) == 0
    n_chunks = per_w // _C

    mesh = plsc.VectorSubcoreMesh(core_axis_name="c", subcore_axis_name="s")

    @functools.partial(
        pl.kernel,
        mesh=mesh,
        compiler_params=pltpu.CompilerParams(use_tc_tiling_on_sc=False),
        out_type=jax.ShapeDtypeStruct((n_tokens, _D), jnp.float32),
        scratch_types=[
            pltpu.VMEM((2, _C), jnp.int32),
            pltpu.VMEM((2, _NUM_HASHES, _C), jnp.int32),
            pltpu.VMEM((2, _NUM_HASHES, _C), jnp.float32),
            pltpu.VMEM((2, _NUM_HASHES, _C, _D), jnp.float32),
            pltpu.VMEM((2, _C, _D), jnp.float32),
            pltpu.SemaphoreType.DMA,
            pltpu.SemaphoreType.DMA,
            pltpu.SemaphoreType.DMA,
            pltpu.SemaphoreType.DMA,
        ],
    )
    def k(tok_hbm, tab_hbm, out_hbm, tok_v, idx_v, sign_v, rows_v, out_v,
          sem_g0, sem_g1, sem_o0, sem_o1):
        i32 = jnp.int32
        wid = lax.axis_index("s") * i32(nc) + lax.axis_index("c")
        wbase = wid * i32(per_w)
        sem_g = (sem_g0, sem_g1)
        sem_o = (sem_o0, sem_o1)

        def stage(g, buf):
            b = i32(buf)
            base = wbase + g * i32(_C)
            pltpu.sync_copy(tok_hbm.at[pl.ds(base, _C)], tok_v.at[b])

            def hash_body(j, carry):
                off = j * i32(_L)
                tv = tok_v[b, pl.ds(off, _L)]
                for i in range(_NUM_HASHES):
                    x = tv * _A_MOD[i] + _B_MOD[i]
                    qf = (x.astype(jnp.float32)
                          * jnp.float32(1.0 / _HASH_VOCAB)
                          + jnp.float32(8388608.0))
                    q = lax.bitcast_convert_type(qf, jnp.int32) - 0x4B000000
                    r = x - q * _HASH_VOCAB
                    r = jnp.where(r < 0, r + _HASH_VOCAB, r)
                    r = jnp.where(r >= _HASH_VOCAB, r - _HASH_VOCAB, r)
                    idx_v[b, i, pl.ds(off, _L)] = r + i * _HASH_VOCAB
                    sgn = ((tv >> i) & 1) * 2 - 1
                    sign_v[b, i, pl.ds(off, _L)] = sgn.astype(jnp.float32)
                return carry

            lax.fori_loop(i32(0), i32(_C // _L), hash_body, i32(0))
            for i in range(_NUM_HASHES):
                pltpu.make_async_copy(
                    tab_hbm.at[idx_v.at[b, i32(i)]],
                    rows_v.at[b, i32(i)],
                    sem_g[buf],
                ).start()

        def wait_gathers(buf):
            b = i32(buf)
            for i in range(_NUM_HASHES):
                pltpu.make_async_copy(
                    tab_hbm.at[idx_v.at[b, i32(i)]],
                    rows_v.at[b, i32(i)],
                    sem_g[buf],
                ).wait()

        def out_copy(g, buf):
            base = wbase + g * i32(_C)
            return pltpu.make_async_copy(
                out_v.at[i32(buf)],
                out_hbm.at[pl.ds(base, _C), :],
                sem_o[buf],
            )

        def accumulate(buf):
            b = i32(buf)

            def acc_body(j, carry):
                t0 = j * i32(_L)
                lo = pl.ds(0, _L)
                hi = pl.ds(_L, _L)
                svs = [sign_v[b, i, pl.ds(t0, _L)]
                       for i in range(_NUM_HASHES)]
                for u in range(_L):
                    t = t0 + i32(u)
                    s0, s1, s2, s3 = (svs[0][u], svs[1][u], svs[2][u],
                                      svs[3][u])
                    out_v[b, t, lo] = (s0 * rows_v[b, 0, t, lo]
                                       + s1 * rows_v[b, 1, t, lo]
                                       + s2 * rows_v[b, 2, t, lo]
                                       + s3 * rows_v[b, 3, t, lo])
                    out_v[b, t, hi] = (s0 * rows_v[b, 0, t, hi]
                                       + s1 * rows_v[b, 1, t, hi]
                                       + s2 * rows_v[b, 2, t, hi]
                                       + s3 * rows_v[b, 3, t, hi])
                return carry

            lax.fori_loop(i32(0), i32(_C // _L), acc_body, i32(0))

        stage(i32(0), 0)

        def pair_body(g2, carry):
            g = g2 * i32(2)
            wait_gathers(0)
            stage(g + i32(1), 1)
            pl.when(g2 > 0)(lambda: out_copy(g, 0).wait())
            accumulate(0)
            out_copy(g, 0).start()
            wait_gathers(1)
            pl.when(g2 < i32(n_chunks // 2 - 1))(
                lambda: stage(g + i32(2), 0))
            pl.when(g2 > 0)(lambda: out_copy(g + i32(1), 1).wait())
            accumulate(1)
            out_copy(g + i32(1), 1).start()
            return carry

        lax.fori_loop(i32(0), i32(n_chunks // 2), pair_body, i32(0))
        out_copy(i32(n_chunks - 2), 0).wait()
        out_copy(i32(n_chunks - 1), 1).wait()

    return k


def kernel(token_ids, tables):
    n = token_ids.shape[0] * token_ids.shape[1]
    tok = token_ids.reshape(n).astype(jnp.int32)
    tab = tables.reshape(_NUM_HASHES * _HASH_VOCAB, _D).astype(jnp.float32)
    out = _make_sc_kernel(n)(tok, tab)
    out = out.reshape(token_ids.shape + (_D,))
    return out.astype(jnp.promote_types(jnp.float32, tables.dtype))


# speedup vs baseline: 2.6755x; 1.0153x over previous
import functools

import jax
import jax.numpy as jnp
from jax import lax
from jax.experimental import pallas as pl
from jax.experimental.pallas import tpu as pltpu
from jax.experimental.pallas import tpu_sc as plsc

_NUM_HASHES = 4
_HASH_VOCAB = 100000
_D = 32
_L = 16

_A_MOD = (7, 11, 13, 17)
_B_MOD = (13, 99996, 99979, 99962)

_C = 256


@functools.cache
def _make_sc_kernel(n_tokens):
    info = plsc.get_sparse_core_info()
    nc, ns = info.num_cores, info.num_subcores
    nw = nc * ns
    per_w = n_tokens // nw
    assert per_w * nw == n_tokens and per_w % (2 * _C) == 0
    n_chunks = per_w // _C

    mesh = plsc.VectorSubcoreMesh(core_axis_name="c", subcore_axis_name="s")

    @functools.partial(
        pl.kernel,
        mesh=mesh,
        compiler_params=pltpu.CompilerParams(use_tc_tiling_on_sc=False),
        out_type=jax.ShapeDtypeStruct((n_tokens, _D), jnp.float32),
        scratch_types=[
            pltpu.VMEM((per_w,), jnp.int32),
            pltpu.VMEM((2, _NUM_HASHES, _C), jnp.int32),
            pltpu.VMEM((2, _NUM_HASHES, _C), jnp.float32),
            pltpu.VMEM((2, _NUM_HASHES, _C, _D), jnp.float32),
            pltpu.VMEM((2, _C, _D), jnp.float32),
            pltpu.SemaphoreType.DMA,
            pltpu.SemaphoreType.DMA,
            pltpu.SemaphoreType.DMA,
            pltpu.SemaphoreType.DMA,
        ],
    )
    def k(tok_hbm, tab_hbm, out_hbm, tok_v, idx_v, sign_v, rows_v, out_v,
          sem_g0, sem_g1, sem_o0, sem_o1):
        i32 = jnp.int32
        wid = lax.axis_index("s") * i32(nc) + lax.axis_index("c")
        wbase = wid * i32(per_w)
        sem_g = (sem_g0, sem_g1)
        sem_o = (sem_o0, sem_o1)

        def stage(g, buf):
            b = i32(buf)
            gof = g * i32(_C)

            def hash_body(j, carry):
                off = j * i32(_L)
                tv = tok_v[pl.ds(gof + off, _L)]
                for i in range(_NUM_HASHES):
                    x = tv * _A_MOD[i] + _B_MOD[i]
                    qf = (x.astype(jnp.float32)
                          * jnp.float32(1.0 / _HASH_VOCAB)
                          + jnp.float32(8388608.0))
                    q = lax.bitcast_convert_type(qf, jnp.int32) - 0x4B000000
                    r = x - q * _HASH_VOCAB
                    r = jnp.where(r < 0, r + _HASH_VOCAB, r)
                    r = jnp.where(r >= _HASH_VOCAB, r - _HASH_VOCAB, r)
                    idx_v[b, i, pl.ds(off, _L)] = r + i * _HASH_VOCAB
                    sgn = ((tv >> i) & 1) * 2 - 1
                    sign_v[b, i, pl.ds(off, _L)] = sgn.astype(jnp.float32)
                return carry

            lax.fori_loop(i32(0), i32(_C // _L), hash_body, i32(0))
            for i in range(_NUM_HASHES):
                pltpu.make_async_copy(
                    tab_hbm.at[idx_v.at[b, i32(i)]],
                    rows_v.at[b, i32(i)],
                    sem_g[buf],
                ).start()

        def wait_gathers(buf):
            b = i32(buf)
            for i in range(_NUM_HASHES):
                pltpu.make_async_copy(
                    tab_hbm.at[idx_v.at[b, i32(i)]],
                    rows_v.at[b, i32(i)],
                    sem_g[buf],
                ).wait()

        def out_copy(g, buf):
            base = wbase + g * i32(_C)
            return pltpu.make_async_copy(
                out_v.at[i32(buf)],
                out_hbm.at[pl.ds(base, _C), :],
                sem_o[buf],
            )

        def accumulate(buf):
            b = i32(buf)

            def acc_body(j, carry):
                t0 = j * i32(_L)
                lo = pl.ds(0, _L)
                hi = pl.ds(_L, _L)
                svs = [sign_v[b, i, pl.ds(t0, _L)]
                       for i in range(_NUM_HASHES)]
                for u in range(_L):
                    t = t0 + i32(u)
                    s0, s1, s2, s3 = (svs[0][u], svs[1][u], svs[2][u],
                                      svs[3][u])
                    out_v[b, t, lo] = (s0 * rows_v[b, 0, t, lo]
                                       + s1 * rows_v[b, 1, t, lo]
                                       + s2 * rows_v[b, 2, t, lo]
                                       + s3 * rows_v[b, 3, t, lo])
                    out_v[b, t, hi] = (s0 * rows_v[b, 0, t, hi]
                                       + s1 * rows_v[b, 1, t, hi]
                                       + s2 * rows_v[b, 2, t, hi]
                                       + s3 * rows_v[b, 3, t, hi])
                return carry

            lax.fori_loop(i32(0), i32(_C // _L), acc_body, i32(0))

        pltpu.sync_copy(tok_hbm.at[pl.ds(wbase, per_w)], tok_v)
        stage(i32(0), 0)

        def pair_body(g2, carry):
            g = g2 * i32(2)
            wait_gathers(0)
            stage(g + i32(1), 1)
            pl.when(g2 > 0)(lambda: out_copy(g, 0).wait())
            accumulate(0)
            out_copy(g, 0).start()
            wait_gathers(1)
            pl.when(g2 < i32(n_chunks // 2 - 1))(
                lambda: stage(g + i32(2), 0))
            pl.when(g2 > 0)(lambda: out_copy(g + i32(1), 1).wait())
            accumulate(1)
            out_copy(g + i32(1), 1).start()
            return carry

        lax.fori_loop(i32(0), i32(n_chunks // 2), pair_body, i32(0))
        out_copy(i32(n_chunks - 2), 0).wait()
        out_copy(i32(n_chunks - 1), 1).wait()

    return k


def kernel(token_ids, tables):
    n = token_ids.shape[0] * token_ids.shape[1]
    tok = token_ids.reshape(n).astype(jnp.int32)
    tab = tables.reshape(_NUM_HASHES * _HASH_VOCAB, _D).astype(jnp.float32)
    out = _make_sc_kernel(n)(tok, tab)
    out = out.reshape(token_ids.shape + (_D,))
    return out.astype(jnp.promote_types(jnp.float32, tables.dtype))
